# Initial kernel scaffold; baseline (speedup 1.0000x reference)
import jax
import jax.numpy as jnp
from jax.experimental import pallas as pl


def kernel(x, W, view_idx):
    raise NotImplementedError("write your pallas kernel here")


# speedup vs baseline: 3.5921x; 3.5921x over previous
import jax
import jax.numpy as jnp
from jax.experimental import pallas as pl
from jax.experimental.pallas import tpu as pltpu


_C_IN = 96
_E = 32
_C_OUT = _C_IN + _E
_HW = 224 * 224
_KSPLIT = 8


def _body(idx_ref, x_ref, w_ref, out_ref):
    out_ref[0, :_C_IN, :] = x_ref[0]
    w = w_ref[idx_ref[0, 0], :]
    out_ref[0, _C_IN:, :] = jnp.broadcast_to(w[:, None], (_E, out_ref.shape[2]))


def kernel(x, W, view_idx):
    B, C, H, Wd = x.shape
    hw = H * Wd
    k = hw // _KSPLIT
    x3 = x.reshape(B, C, hw)
    idx = jnp.asarray(view_idx, jnp.int32).reshape(1)

    out3 = pl.pallas_call(
        _body,
        grid=(B, _KSPLIT),
        in_specs=[
            pl.BlockSpec((1, 1), lambda b, j: (0, 0), memory_space=pltpu.SMEM),
            pl.BlockSpec((1, C, k), lambda b, j: (b, 0, j)),
            pl.BlockSpec((W.shape[0], W.shape[1]), lambda b, j: (0, 0)),
        ],
        out_specs=pl.BlockSpec((1, _C_OUT, k), lambda b, j: (b, 0, j)),
        out_shape=jax.ShapeDtypeStruct((B, _C_OUT, hw), x.dtype),
    )(idx.reshape(1, 1), x3, W)
    return out3.reshape(B, _C_OUT, H, Wd)
